# baseline (speedup 1.0000x reference)
import functools

import jax
import jax.numpy as jnp
import numpy as np
from jax import lax
from jax.experimental import pallas as pl
from jax.experimental.pallas import tpu as pltpu
from jax.experimental.pallas import tpu_sc as plsc

N = 10000
NPAD = 10240
D = 128
E = 160000
NCORE = 2
NSUB = 16
CH = 96
EPT = 10368
NCHUNK = EPT // CH
EPC = NSUB * EPT
RPT = NPAD // NSUB
WB = 16
BLK = 1024
NEG = -1e30


def _proj_body(x_ref, ws_ref, avs_ref, wd_ref, avd_ref, h_ref, aa_ref, mx_ref):
    x = x_ref[0]
    dn = (((1,), (1,)), ((), ()))
    ws_row = lax.dot_general(avs_ref[0], ws_ref[0], dn,
                             preferred_element_type=jnp.float32)
    wd_row = lax.dot_general(avd_ref[0], wd_ref[0], dn,
                             preferred_element_type=jnp.float32)
    h_ref[...] = jnp.dot(
        x, ws_ref[0], preferred_element_type=jnp.float32
    ).astype(jnp.bfloat16)
    a_s = jnp.sum(x * ws_row, axis=1, keepdims=True)
    a_d = jnp.sum(x * wd_row, axis=1, keepdims=True)
    aa_ref[...] = jnp.concatenate(
        [a_s, a_d, jnp.zeros((x.shape[0], 14), jnp.float32)], axis=1)
    cur = jnp.concatenate([
        jnp.full((1, 128), jnp.max(a_s), jnp.float32),
        jnp.full((1, 128), jnp.max(a_d), jnp.float32),
        jnp.full((6, 128), NEG, jnp.float32),
    ], axis=0)

    @pl.when(pl.program_id(1) == 0)
    def _():
        mx_ref[0] = cur

    @pl.when(pl.program_id(1) != 0)
    def _():
        mx_ref[0] = jnp.maximum(mx_ref[0], cur)


def _tc_proj(x2, Ws2, avs2, Wd2, avd2):
    return pl.pallas_call(
        _proj_body,
        grid=(2, NPAD // BLK),
        in_specs=[
            pl.BlockSpec((1, BLK, D), lambda t, i: (t, i, 0)),
            pl.BlockSpec((1, D, D), lambda t, i: (t, 0, 0)),
            pl.BlockSpec((1, 1, D), lambda t, i: (t, 0, 0)),
            pl.BlockSpec((1, D, D), lambda t, i: (t, 0, 0)),
            pl.BlockSpec((1, 1, D), lambda t, i: (t, 0, 0)),
        ],
        out_specs=[
            pl.BlockSpec((BLK, D), lambda t, i: (t * (NPAD // BLK) + i, 0)),
            pl.BlockSpec((BLK, 16), lambda t, i: (t * (NPAD // BLK) + i, 0)),
            pl.BlockSpec((1, 8, 128), lambda t, i: (t, 0, 0)),
        ],
        out_shape=[
            jax.ShapeDtypeStruct((2 * NPAD, D), jnp.bfloat16),
            jax.ShapeDtypeStruct((2 * NPAD, 16), jnp.float32),
            jax.ShapeDtypeStruct((2, 8, 128), jnp.float32),
        ],
    )(x2.reshape(2, NPAD, D), Ws2, avs2.reshape(2, 1, D), Wd2,
      avd2.reshape(2, 1, D))


def _sc_edge(H, AS, AD, edges, g2, b2, zf, zv):
    mesh = plsc.VectorSubcoreMesh(core_axis_name="c", subcore_axis_name="s",
                                  num_cores=NCORE, num_subcores=NSUB)

    @functools.partial(
        pl.kernel,
        out_type=jax.ShapeDtypeStruct((NCORE * NPAD, D), jnp.float32),
        mesh=mesh,
        compiler_params=pltpu.CompilerParams(needs_layout_passes=False,
                                             use_tc_tiling_on_sc=False),
        scratch_types=[
            pltpu.VMEM_SHARED((NPAD, D), jnp.float32),
            pltpu.VMEM_SHARED((NPAD,), jnp.float32),
            pltpu.VMEM((3, 2, CH), jnp.int32),
            pltpu.VMEM((3, CH), jnp.int32),
            pltpu.VMEM((2, CH), jnp.int32),
            pltpu.VMEM((3, CH), jnp.float32),
            pltpu.VMEM((3, CH), jnp.float32),
            pltpu.VMEM((2, CH), jnp.float32),
            pltpu.VMEM((3, CH, D // 2), jnp.float32),
            pltpu.VMEM((2, CH, D), jnp.float32),
            pltpu.VMEM((WB, D), jnp.float32),
            pltpu.VMEM((16,), jnp.float32),
            pltpu.VMEM((D,), jnp.float32),
            pltpu.VMEM((WB,), jnp.float32),
            pltpu.SemaphoreType.DMA,
            pltpu.SemaphoreType.DMA,
            pltpu.SemaphoreType.DMA,
            pltpu.SemaphoreType.DMA,
            pltpu.SemaphoreType.DMA,
            pltpu.SemaphoreType.DMA,
            pltpu.SemaphoreType.DMA,
            pltpu.SemaphoreType.DMA,
        ],
    )
    def k(h_hbm, as_hbm, ad_hbm, e_hbm, g_hbm, b_hbm, zf_hbm,
          zv_hbm, out, acc, dacc, idx, idxg, idxd_s, asv, adv, pv, rows,
          scaled, nbuf, gv, bv, dnv, gs0, gs1, gs2, ss0, ss1, is0, is1, is2):
        c = lax.axis_index("c")
        s = lax.axis_index("s")
        gsem = (gs0, gs1, gs2)
        ssem = (ss0, ss1)
        isem = (is0, is1, is2)
        ro = s * RPT
        pltpu.sync_copy(zf_hbm.at[pl.ds(ro, RPT)], acc.at[pl.ds(ro, RPT)])
        pltpu.sync_copy(zv_hbm.at[pl.ds(ro, RPT)], dacc.at[pl.ds(ro, RPT)])
        pltpu.sync_copy(g_hbm.at[c], gv)
        pltpu.sync_copy(b_hbm.at[c], bv)
        plsc.subcore_barrier()
        cbase = (c * NSUB + s) * NCHUNK
        coff = (1 - c) * NPAD

        def starti(kk, b):
            pltpu.async_copy(e_hbm.at[cbase + kk], idx.at[b], isem[b])

        def waiti(b):
            pltpu.make_async_copy(e_hbm.at[0], idx.at[b], isem[b]).wait()

        def startg(b):
            for j in range(CH // 16):
                sl = pl.ds(16 * j, 16)
                idxg[b, sl] = idx[b, 1, sl] + coff
            pltpu.async_copy(as_hbm.at[idx.at[b, 0]], asv.at[b], gsem[b])
            pltpu.async_copy(ad_hbm.at[idxg.at[b]], adv.at[b], gsem[b])
            pltpu.async_copy(h_hbm.at[idx.at[b, 0]], rows.at[b], gsem[b])

        def waitg(b):
            pltpu.make_async_copy(as_hbm.at[pl.ds(0, CH)], asv.at[b],
                                  gsem[b]).wait()
            pltpu.make_async_copy(ad_hbm.at[pl.ds(0, CH)], adv.at[b],
                                  gsem[b]).wait()
            pltpu.make_async_copy(h_hbm.at[pl.ds(0, CH)], rows.at[b],
                                  gsem[b]).wait()

        def starts(sb):
            pltpu.async_copy(pv.at[sb], dacc.at[idxd_s.at[sb]], ssem[sb],
                             add=True)
            pltpu.async_copy(scaled.at[sb], acc.at[idxd_s.at[sb]], ssem[sb],
                             add=True)

        def waits(sb):
            pltpu.make_async_copy(zv_hbm.at[pl.ds(0, CH)], pv.at[sb],
                                  ssem[sb]).wait()
            pltpu.make_async_copy(zf_hbm.at[pl.ds(0, CH)], scaled.at[sb],
                                  ssem[sb]).wait()

        starti(0, 0)
        waiti(0)
        startg(0)
        starti(1, 1)

        def ring(g, carry):
            k0 = 6 * g
            for u in range(6):
                kk = k0 + u
                b = u % 3
                sb = u % 2
                bg = (b + 1) % 3
                bn = (b + 2) % 3
                waitg(b)

                @pl.when(kk + 1 < NCHUNK)
                def _(bg=bg):
                    waiti(bg)
                    startg(bg)

                @pl.when(kk >= 2)
                def _(sb=sb):
                    waits(sb)

                for j in range(CH // 16):
                    sl = pl.ds(16 * j, 16)
                    idxd_s[sb, sl] = idx[b, 1, sl]

                gvec = gv[...]
                for j in range(CH // 16):
                    sl = pl.ds(16 * j, 16)
                    a = asv[b, sl] + adv[b, sl]
                    a = jnp.where(a >= 0.0, a, a * 0.2)
                    pv[sb, sl] = jnp.exp(a - gvec)

                def srow(e, cc, b=b, sb=sb):
                    pe = plsc.load_gather(pv.at[sb],
                                          [jnp.full((16,), e, jnp.int32)])
                    for j in range(4):
                        rw = rows[b, e, pl.ds(16 * j, 16)]
                        rb = plsc.bitcast(rw, jnp.bfloat16)
                        lo, hi = plsc.unpack(
                            rb, format=plsc.PackFormat.INTERLEAVED)
                        scaled[sb, e, pl.ds(32 * j, 16)] = lo * pe
                        scaled[sb, e, pl.ds(32 * j + 16, 16)] = hi * pe
                    return cc

                lax.fori_loop(0, CH, srow, 0, unroll=4)
                starts(sb)

                @pl.when(kk + 2 < NCHUNK)
                def _(kk=kk, bn=bn):
                    starti(kk + 2, bn)

            return carry

        lax.fori_loop(0, NCHUNK // 6, ring, 0)
        waits(0)
        waits(1)
        plsc.subcore_barrier()

        def wblock(b, carry):
            rbase = ro + b * WB
            pltpu.sync_copy(dacc.at[pl.ds(rbase, WB)], dnv)
            pltpu.sync_copy(acc.at[pl.ds(rbase, WB)], nbuf)

            def nrow(e, cc):
                d = plsc.load_gather(
                    dnv, [jnp.full((16,), e, jnp.int32)])
                q = 1.0 / (d + 1e-16)
                for j in range(8):
                    sl = pl.ds(16 * j, 16)
                    nbuf[e, sl] = jnp.maximum(nbuf[e, sl] * q + bv[sl], 0.0)
                return cc

            lax.fori_loop(0, WB, nrow, 0, unroll=4)
            pltpu.sync_copy(nbuf, out.at[pl.ds(coff + rbase, WB)])
            return carry

        lax.fori_loop(0, RPT // WB, wblock, 0)

    return k(H, AS, AD, edges, g2, b2, zf, zv)


def kernel(x_user, x_item, edge_index_user_item, edge_index_item_user,
           W_src, W_dst, att_src, att_dst, bias):
    def _pad_edges(arr, fill):
        a = arr.astype(jnp.int32).reshape(NSUB, E // NSUB)
        a = jnp.pad(a, ((0, 0), (0, EPT - E // NSUB)), constant_values=fill)
        return a.reshape(EPC)

    src0 = _pad_edges(edge_index_user_item[0], 0)
    dst0 = _pad_edges(edge_index_user_item[1], NPAD - 1)
    src1 = _pad_edges(edge_index_item_user[0], 0)
    dst1 = _pad_edges(edge_index_item_user[1], NPAD - 1)
    edges = jnp.stack([
        jnp.stack([src0.reshape(NSUB, NCHUNK, CH),
                   dst0.reshape(NSUB, NCHUNK, CH)], axis=2),
        jnp.stack([(src1 + NPAD).reshape(NSUB, NCHUNK, CH),
                   dst1.reshape(NSUB, NCHUNK, CH)], axis=2),
    ]).reshape(NCORE * NSUB * NCHUNK, 2, CH)
    pad = ((0, NPAD - N), (0, 0))
    x2 = jnp.concatenate([jnp.pad(x_user, pad), jnp.pad(x_item, pad)], axis=0)
    zf = jnp.zeros((NPAD, D), jnp.float32)
    zv = jnp.zeros((NPAD,), jnp.float32)
    perm = np.empty((D,), np.int32)
    for j in range(D // 32):
        for t in range(16):
            perm[32 * j + 2 * t] = 32 * j + t
            perm[32 * j + 2 * t + 1] = 32 * j + 16 + t
    for l in range(2):
        H, aa, mx = _tc_proj(x2, W_src[l][:, :, perm], att_src[l][:, perm],
                             W_dst[l, ::-1], att_dst[l, ::-1])
        g0 = mx[0, 0, 0] + mx[1, 1, 0]
        g1 = mx[1, 0, 0] + mx[0, 1, 0]
        g0 = jnp.where(g0 >= 0.0, g0, 0.2 * g0)
        g1 = jnp.where(g1 >= 0.0, g1, 0.2 * g1)
        g2 = jnp.stack([jnp.full((16,), g0), jnp.full((16,), g1)])
        b2 = jnp.stack([bias[l, 0], bias[l, 1]])
        Hp = lax.bitcast_convert_type(
            H.reshape(2 * NPAD, D // 2, 2), jnp.float32)
        x2 = _sc_edge(Hp, aa[:, 0], aa[:, 1], edges, g2, b2, zf, zv)
    return x2[:N], x2[NPAD:NPAD + N]


# speedup vs baseline: 1.8545x; 1.8545x over previous
import functools

import jax
import jax.numpy as jnp
from jax import lax
from jax.experimental import pallas as pl
from jax.experimental.pallas import tpu as pltpu
from jax.experimental.pallas import tpu_sc as plsc

N = 10000
NPAD = 10240
D = 128
E = 160000
NCORE = 2
NSUB = 16
CH = 96
EPT = 10080
NCHUNK = EPT // CH
EPC = NSUB * EPT
RPT = NPAD // NSUB
WB = 32
BLK = 1024
NEG = -1e30


def _proj_body(x_ref, ws_ref, avs_ref, wd_ref, avd_ref, h_ref, aa_ref, mx_ref):
    x = x_ref[0]
    dn = (((1,), (1,)), ((), ()))
    ws_row = lax.dot_general(avs_ref[0], ws_ref[0], dn,
                             preferred_element_type=jnp.float32)
    wd_row = lax.dot_general(avd_ref[0], wd_ref[0], dn,
                             preferred_element_type=jnp.float32)
    h_ref[...] = jnp.dot(x, ws_ref[0], preferred_element_type=jnp.float32)
    a_s = jnp.sum(x * ws_row, axis=1, keepdims=True)
    a_d = jnp.sum(x * wd_row, axis=1, keepdims=True)
    aa_ref[...] = jnp.concatenate(
        [a_s, a_d, jnp.zeros((x.shape[0], 14), jnp.float32)], axis=1)
    cur = jnp.concatenate([
        jnp.full((1, 128), jnp.max(a_s), jnp.float32),
        jnp.full((1, 128), jnp.max(a_d), jnp.float32),
        jnp.full((6, 128), NEG, jnp.float32),
    ], axis=0)

    @pl.when(pl.program_id(1) == 0)
    def _():
        mx_ref[0] = cur

    @pl.when(pl.program_id(1) != 0)
    def _():
        mx_ref[0] = jnp.maximum(mx_ref[0], cur)


def _tc_proj(x2, Ws2, avs2, Wd2, avd2):
    return pl.pallas_call(
        _proj_body,
        grid=(2, NPAD // BLK),
        in_specs=[
            pl.BlockSpec((1, BLK, D), lambda t, i: (t, i, 0)),
            pl.BlockSpec((1, D, D), lambda t, i: (t, 0, 0)),
            pl.BlockSpec((1, 1, D), lambda t, i: (t, 0, 0)),
            pl.BlockSpec((1, D, D), lambda t, i: (t, 0, 0)),
            pl.BlockSpec((1, 1, D), lambda t, i: (t, 0, 0)),
        ],
        out_specs=[
            pl.BlockSpec((BLK, D), lambda t, i: (t * (NPAD // BLK) + i, 0)),
            pl.BlockSpec((BLK, 16), lambda t, i: (t * (NPAD // BLK) + i, 0)),
            pl.BlockSpec((1, 8, 128), lambda t, i: (t, 0, 0)),
        ],
        out_shape=[
            jax.ShapeDtypeStruct((2 * NPAD, D), jnp.float32),
            jax.ShapeDtypeStruct((2 * NPAD, 16), jnp.float32),
            jax.ShapeDtypeStruct((2, 8, 128), jnp.float32),
        ],
    )(x2.reshape(2, NPAD, D), Ws2, avs2.reshape(2, 1, D), Wd2,
      avd2.reshape(2, 1, D))


def _sc_edge(H, AS, AD, edges, g2, b2, zf, zv):
    mesh = plsc.VectorSubcoreMesh(core_axis_name="c", subcore_axis_name="s",
                                  num_cores=NCORE, num_subcores=NSUB)

    @functools.partial(
        pl.kernel,
        out_type=jax.ShapeDtypeStruct((NCORE * NPAD, D), jnp.float32),
        mesh=mesh,
        compiler_params=pltpu.CompilerParams(needs_layout_passes=False),
        scratch_types=[
            pltpu.VMEM_SHARED((NPAD, D), jnp.float32),
            pltpu.VMEM_SHARED((NPAD,), jnp.float32),
            pltpu.VMEM((3, 2, CH), jnp.int32),
            pltpu.VMEM((3, CH), jnp.int32),
            pltpu.VMEM((3, CH), jnp.float32),
            pltpu.VMEM((3, CH), jnp.float32),
            pltpu.VMEM((3, CH), jnp.float32),
            pltpu.VMEM((3, CH, D), jnp.float32),
            pltpu.VMEM((WB, D), jnp.float32),
            pltpu.VMEM((16,), jnp.float32),
            pltpu.VMEM((D,), jnp.float32),
            pltpu.VMEM((WB,), jnp.float32),
            pltpu.SemaphoreType.DMA,
            pltpu.SemaphoreType.DMA,
            pltpu.SemaphoreType.DMA,
            pltpu.SemaphoreType.DMA,
            pltpu.SemaphoreType.DMA,
            pltpu.SemaphoreType.DMA,
            pltpu.SemaphoreType.DMA,
            pltpu.SemaphoreType.DMA,
            pltpu.SemaphoreType.DMA,
        ],
    )
    def k(h_hbm, as_hbm, ad_hbm, e_hbm, g_hbm, b_hbm, zf_hbm,
          zv_hbm, out, acc, dacc, idx, idxg, asv, adv, pv, rows,
          nbuf, gv, bv, dnv, gs0, gs1, gs2, ss0, ss1, ss2, is0, is1, is2):
        c = lax.axis_index("c")
        s = lax.axis_index("s")
        gsem = (gs0, gs1, gs2)
        ssem = (ss0, ss1, ss2)
        isem = (is0, is1, is2)
        ro = s * RPT
        pltpu.sync_copy(zf_hbm.at[pl.ds(ro, RPT)], acc.at[pl.ds(ro, RPT)])
        pltpu.sync_copy(zv_hbm.at[pl.ds(ro, RPT)], dacc.at[pl.ds(ro, RPT)])
        pltpu.sync_copy(g_hbm.at[c], gv)
        pltpu.sync_copy(b_hbm.at[c], bv)
        plsc.subcore_barrier()
        cbase = (c * NSUB + s) * NCHUNK
        coff = (1 - c) * NPAD

        def starti(kk, b):
            pltpu.async_copy(e_hbm.at[cbase + kk], idx.at[b], isem[b])

        def waiti(b):
            pltpu.make_async_copy(e_hbm.at[0], idx.at[b], isem[b]).wait()

        def startg(b):
            for j in range(CH // 16):
                sl = pl.ds(16 * j, 16)
                idxg[b, sl] = idx[b, 1, sl] + coff
            pltpu.async_copy(as_hbm.at[idx.at[b, 0]], asv.at[b], gsem[b])
            pltpu.async_copy(ad_hbm.at[idxg.at[b]], adv.at[b], gsem[b])
            pltpu.async_copy(h_hbm.at[idx.at[b, 0]], rows.at[b], gsem[b])

        def waitg(b):
            pltpu.make_async_copy(as_hbm.at[pl.ds(0, CH)], asv.at[b],
                                  gsem[b]).wait()
            pltpu.make_async_copy(ad_hbm.at[pl.ds(0, CH)], adv.at[b],
                                  gsem[b]).wait()
            pltpu.make_async_copy(h_hbm.at[pl.ds(0, CH)], rows.at[b],
                                  gsem[b]).wait()

        def starts(b):
            pltpu.async_copy(pv.at[b], dacc.at[idx.at[b, 1]], ssem[b],
                             add=True)
            pltpu.async_copy(rows.at[b], acc.at[idx.at[b, 1]], ssem[b],
                             add=True)

        def waits(b):
            pltpu.make_async_copy(zv_hbm.at[pl.ds(0, CH)], pv.at[b],
                                  ssem[b]).wait()
            pltpu.make_async_copy(zf_hbm.at[pl.ds(0, CH)], rows.at[b],
                                  ssem[b]).wait()

        starti(0, 0)
        waiti(0)
        startg(0)
        starti(1, 1)

        def ring(g, carry):
            k0 = 3 * g
            for b in range(3):
                kk = k0 + b
                bg = (b + 1) % 3
                bn = (b + 2) % 3
                waitg(b)

                @pl.when(kk + 1 < NCHUNK)
                def _(bg=bg):
                    waiti(bg)
                    startg(bg)

                gvec = gv[...]
                for j in range(CH // 16):
                    sl = pl.ds(16 * j, 16)
                    a = asv[b, sl] + adv[b, sl]
                    a = jnp.where(a >= 0.0, a, a * 0.2)
                    pv[b, sl] = jnp.exp(a - gvec)

                def srow(e, cc, b=b):
                    pe = plsc.load_gather(pv.at[b],
                                          [jnp.full((16,), e, jnp.int32)])
                    for j in range(8):
                        sl = pl.ds(16 * j, 16)
                        rows[b, e, sl] = rows[b, e, sl] * pe
                    return cc

                lax.fori_loop(0, CH, srow, 0, unroll=4)
                starts(b)
                kn = kk + 2

                @pl.when(kn < NCHUNK)
                def _(kk=kk, kn=kn, bn=bn):
                    @pl.when(kk >= 1)
                    def _():
                        waits(bn)

                    starti(kn, bn)

            return carry

        lax.fori_loop(0, NCHUNK // 3, ring, 0)
        waits(0)
        waits(1)
        waits(2)
        plsc.subcore_barrier()

        def wblock(b, carry):
            rbase = ro + b * WB
            pltpu.sync_copy(dacc.at[pl.ds(rbase, WB)], dnv)
            pltpu.sync_copy(acc.at[pl.ds(rbase, WB)], nbuf)

            def nrow(e, cc):
                d = plsc.load_gather(
                    dnv, [jnp.full((16,), e, jnp.int32)])
                q = 1.0 / (d + 1e-16)
                for j in range(8):
                    sl = pl.ds(16 * j, 16)
                    nbuf[e, sl] = jnp.maximum(nbuf[e, sl] * q + bv[sl], 0.0)
                return cc

            lax.fori_loop(0, WB, nrow, 0, unroll=4)
            pltpu.sync_copy(nbuf, out.at[pl.ds(coff + rbase, WB)])
            return carry

        lax.fori_loop(0, RPT // WB, wblock, 0)

    return k(H, AS, AD, edges, g2, b2, zf, zv)


def kernel(x_user, x_item, edge_index_user_item, edge_index_item_user,
           W_src, W_dst, att_src, att_dst, bias):
    def _pad_edges(arr, fill):
        a = arr.astype(jnp.int32).reshape(NSUB, E // NSUB)
        a = jnp.pad(a, ((0, 0), (0, EPT - E // NSUB)), constant_values=fill)
        return a.reshape(EPC)

    src0 = _pad_edges(edge_index_user_item[0], 0)
    dst0 = _pad_edges(edge_index_user_item[1], NPAD - 1)
    src1 = _pad_edges(edge_index_item_user[0], 0)
    dst1 = _pad_edges(edge_index_item_user[1], NPAD - 1)
    edges = jnp.stack([
        jnp.stack([src0.reshape(NSUB, NCHUNK, CH),
                   dst0.reshape(NSUB, NCHUNK, CH)], axis=2),
        jnp.stack([(src1 + NPAD).reshape(NSUB, NCHUNK, CH),
                   dst1.reshape(NSUB, NCHUNK, CH)], axis=2),
    ]).reshape(NCORE * NSUB * NCHUNK, 2, CH)
    pad = ((0, NPAD - N), (0, 0))
    x2 = jnp.concatenate([jnp.pad(x_user, pad), jnp.pad(x_item, pad)], axis=0)
    zf = jnp.zeros((NPAD, D), jnp.float32)
    zv = jnp.zeros((NPAD,), jnp.float32)
    for l in range(2):
        H, aa, mx = _tc_proj(x2, W_src[l], att_src[l],
                             W_dst[l, ::-1], att_dst[l, ::-1])
        g0 = mx[0, 0, 0] + mx[1, 1, 0]
        g1 = mx[1, 0, 0] + mx[0, 1, 0]
        g0 = jnp.where(g0 >= 0.0, g0, 0.2 * g0)
        g1 = jnp.where(g1 >= 0.0, g1, 0.2 * g1)
        g2 = jnp.stack([jnp.full((16,), g0), jnp.full((16,), g1)])
        b2 = jnp.stack([bias[l, 0], bias[l, 1]])
        x2 = _sc_edge(H, aa[:, 0], aa[:, 1], edges, g2, b2, zf, zv)
    return x2[:N], x2[NPAD:NPAD + N]
